# baseline (speedup 1.0000x reference)
import functools

import jax
import jax.numpy as jnp
from jax import lax
from jax.experimental import pallas as pl
from jax.experimental.pallas import tpu as pltpu
from jax.experimental.pallas import tpu_sc as plsc

_N = 10000
_E = 160000
_DF = 128
_DE = 16
_DIM = 32
_HID = 32
_NG = 64

_NC = 2
_NS = 16
_NW = _NC * _NS
_LCH = 128
_NCH = _E // _LCH
_NB_NODE = 5
_MB = _N // _NB_NODE
_EB = 4000
_NEB = _E // _EB


def _hcat_body(x_ref, w0_ref, b0_ref, bmat_ref, o_ref):
    h0 = jnp.dot(x_ref[...], w0_ref[...], preferred_element_type=jnp.float32)
    h0 = jnp.maximum(h0 + b0_ref[...], 0.0)
    o_ref[:, :_DIM] = h0
    o_ref[:, _DIM:] = jnp.dot(h0, bmat_ref[...], preferred_element_type=jnp.float32)


def _run_hcat(x, W0, b0, Bmat):
    return pl.pallas_call(
        _hcat_body,
        grid=(_NB_NODE,),
        in_specs=[
            pl.BlockSpec((_MB, _DF), lambda i: (i, 0)),
            pl.BlockSpec((_DF, _DIM), lambda i: (0, 0)),
            pl.BlockSpec((1, _DIM), lambda i: (0, 0)),
            pl.BlockSpec((_DIM, _HID), lambda i: (0, 0)),
        ],
        out_specs=pl.BlockSpec((_MB, 2 * _DIM), lambda i: (i, 0)),
        out_shape=jax.ShapeDtypeStruct((_N, 2 * _DIM), jnp.float32),
    )(x, W0, b0.reshape(1, _DIM), Bmat)


def _gather_body(hcat_hbm, src_hbm, xj_hbm, idx_v, rows_v, sem):
    cid = lax.axis_index("c")
    sid = lax.axis_index("s")
    wid = sid * _NC + cid
    nj = (_NCH // _NW) + (wid < (_NCH % _NW)).astype(jnp.int32)

    def step(j, carry):
        c = wid + j * _NW
        base = pl.multiple_of(c * _LCH, 8)
        pltpu.sync_copy(src_hbm.at[pl.ds(base, _LCH)], idx_v)
        pltpu.async_copy(hcat_hbm.at[idx_v], rows_v, sem).wait()
        pltpu.sync_copy(rows_v, xj_hbm.at[pl.ds(base, _LCH)])
        return carry

    lax.fori_loop(0, nj, step, 0)


def _run_gather(hcat, src):
    mesh = plsc.VectorSubcoreMesh(
        core_axis_name="c", subcore_axis_name="s", num_cores=_NC, num_subcores=_NS
    )
    f = functools.partial(
        pl.kernel,
        out_type=jax.ShapeDtypeStruct((_E, 2 * _DIM), jnp.float32),
        mesh=mesh,
        scratch_types=[
            pltpu.VMEM((_LCH,), jnp.int32),
            pltpu.VMEM((_LCH, 2 * _DIM), jnp.float32),
            pltpu.SemaphoreType.DMA,
        ],
        compiler_params=pltpu.CompilerParams(use_tc_tiling_on_sc=False),
    )(_gather_body)
    return f(hcat, src)


_MSG_DT = jnp.bfloat16


def _msg_body(xj_ref, ea_ref, wcT_ref, o_ref):
    xjP = jnp.concatenate([xj_ref[:, : 2 * _DIM], xj_ref[:, 2 * _DIM :]], axis=0)
    xjT = xjP[:, :_DIM].astype(_MSG_DT).T
    eaT = jnp.concatenate(
        [ea_ref[:, 0, 0, :], ea_ref[:, 0, 1, :]], axis=1
    ).astype(_MSG_DT)
    fT = jnp.concatenate(
        [eaT[k : k + 1, :] * xjT for k in range(_DE)], axis=0
    )
    msgT = jnp.dot(wcT_ref[...], fT, preferred_element_type=jnp.float32)
    msg = msgT.T + xjP[:, _DIM:]
    q = _EB // 4
    o_ref[...] = jnp.concatenate(
        [msg[g * q : (g + 1) * q] for g in range(4)], axis=1
    )


def _run_msg(xj128, eaPT, WcT):
    return pl.pallas_call(
        _msg_body,
        grid=(_NEB,),
        in_specs=[
            pl.BlockSpec((_EB // 2, 128), lambda i: (i, 0)),
            pl.BlockSpec((_DE, 1, 2, _EB // 2), lambda i: (0, i, 0, 0)),
            pl.BlockSpec((_HID, _DE * _DIM), lambda i: (0, 0)),
        ],
        out_specs=pl.BlockSpec((_EB // 4, 4 * _HID), lambda i: (i, 0)),
        out_shape=jax.ShapeDtypeStruct((_E // 4, 4 * _HID), jnp.float32),
    )(xj128, eaPT, WcT)


_ZROWS = 125


def _scatter_body(msg_hbm, dst_hbm, out_hbm, idx_v, msg_v, zero_v, shared, sem):
    cid = lax.axis_index("c")
    sid = lax.axis_index("s")
    wid = sid * _NC + cid
    stripe = _N // _NS

    z16 = jnp.zeros((16,), jnp.float32)

    def zrow(i, c):
        zero_v[i, pl.ds(0, 16)] = z16
        zero_v[i, pl.ds(16, 16)] = z16
        return c

    lax.fori_loop(0, _ZROWS, zrow, 0)

    def zcopy(t, c):
        pltpu.sync_copy(zero_v, shared.at[pl.ds(sid * stripe + t * _ZROWS, _ZROWS)])
        return c

    lax.fori_loop(0, stripe // _ZROWS, zcopy, 0)
    plsc.subcore_barrier()

    nj = (_NCH // _NW) + (wid < (_NCH % _NW)).astype(jnp.int32)

    def step(j, carry):
        c = wid + j * _NW
        base = pl.multiple_of(c * _LCH, 8)
        pltpu.sync_copy(dst_hbm.at[pl.ds(base, _LCH)], idx_v)
        pltpu.sync_copy(msg_hbm.at[pl.ds(base, _LCH)], msg_v)
        pltpu.sync_copy(msg_v, shared.at[idx_v], add=True)
        return carry

    lax.fori_loop(0, nj, step, 0)
    plsc.subcore_barrier()

    pltpu.sync_copy(
        shared.at[pl.ds(sid * stripe, stripe)],
        out_hbm.at[cid, pl.ds(sid * stripe, stripe)],
    )


def _run_scatter(msg, dst):
    mesh = plsc.VectorSubcoreMesh(
        core_axis_name="c", subcore_axis_name="s", num_cores=_NC, num_subcores=_NS
    )
    f = functools.partial(
        pl.kernel,
        out_type=jax.ShapeDtypeStruct((_NC, _N, _HID), jnp.float32),
        mesh=mesh,
        scratch_types=[
            pltpu.VMEM((_LCH,), jnp.int32),
            pltpu.VMEM((_LCH, _HID), jnp.float32),
            pltpu.VMEM((_ZROWS, _HID), jnp.float32),
            pltpu.VMEM_SHARED((_N, _HID), jnp.float32),
            pltpu.SemaphoreType.DMA,
        ],
        compiler_params=pltpu.CompilerParams(use_tc_tiling_on_sc=False),
    )(_scatter_body)
    return f(msg, dst)


def _final_body(hcat_ref, a0_ref, a1_ref, b3_ref, root_ref, bias_ref,
                w1_ref, b1_ref, w2_ref, b2_ref, o_ref, u_acc):
    i = pl.program_id(0)
    h0 = hcat_ref[:, :_DIM]
    h = a0_ref[0] + a1_ref[0] + jnp.dot(h0, root_ref[...], preferred_element_type=jnp.float32)
    h = jnp.maximum(h + bias_ref[...], 0.0)
    bids = b3_ref[0]
    oh = (lax.broadcasted_iota(jnp.int32, (_NG, 1), 0) == bids).astype(jnp.float32)
    part = jnp.dot(oh, h, preferred_element_type=jnp.float32)

    @pl.when(i == 0)
    def _():
        u_acc[...] = part

    @pl.when(i > 0)
    def _():
        u_acc[...] += part

    @pl.when(i == pl.num_programs(0) - 1)
    def _():
        u = u_acc[...]
        o1 = jnp.dot(u, w1_ref[...], preferred_element_type=jnp.float32)
        o1 = jnp.maximum(o1 + b1_ref[...], 0.0)
        o_ref[...] = jnp.dot(o1, w2_ref[...], preferred_element_type=jnp.float32) + b2_ref[...]


def _run_final(hcat, aggr2, batch, root, bias, W1, b1, W2, b2):
    batch3 = batch.reshape(_NB_NODE, 1, _MB)
    return pl.pallas_call(
        _final_body,
        grid=(_NB_NODE,),
        in_specs=[
            pl.BlockSpec((_MB, 2 * _DIM), lambda i: (i, 0)),
            pl.BlockSpec((1, _MB, _HID), lambda i: (0, i, 0)),
            pl.BlockSpec((1, _MB, _HID), lambda i: (1, i, 0)),
            pl.BlockSpec((1, 1, _MB), lambda i: (i, 0, 0)),
            pl.BlockSpec((_DIM, _HID), lambda i: (0, 0)),
            pl.BlockSpec((1, _HID), lambda i: (0, 0)),
            pl.BlockSpec((_HID, 16), lambda i: (0, 0)),
            pl.BlockSpec((1, 16), lambda i: (0, 0)),
            pl.BlockSpec((16, 1), lambda i: (0, 0)),
            pl.BlockSpec((1, 1), lambda i: (0, 0)),
        ],
        out_specs=pl.BlockSpec((_NG, 1), lambda i: (0, 0)),
        out_shape=jax.ShapeDtypeStruct((_NG, 1), jnp.float32),
        scratch_shapes=[pltpu.VMEM((_NG, _HID), jnp.float32)],
    )(hcat, aggr2, aggr2, batch3, root, bias.reshape(1, _HID),
      W1, b1.reshape(1, 16), W2, b2.reshape(1, 1))


def kernel(x, edge_index, edge_attr, batch, W0, b0, We, be, root, bias, W1, b1, W2, b2):
    src = edge_index[0]
    dst = edge_index[1]
    Bmat = be.reshape(_DIM, _HID)
    WcT = We.reshape(_DE * _DIM, _HID).T.astype(_MSG_DT)

    eaPT = edge_attr.reshape(_NEB, _EB // 2, 2, _DE).transpose(3, 0, 2, 1)
    dstP = (
        dst.reshape(_NEB, 2, _EB // 4, 2).transpose(0, 2, 3, 1).reshape(-1)
    )

    hcat = _run_hcat(x, W0, b0, Bmat)
    xj = _run_gather(hcat, src)
    msg128 = _run_msg(xj.reshape(_E // 2, 128), eaPT, WcT)
    aggr2 = _run_scatter(msg128.reshape(_E, _HID), dstP)
    o = _run_final(hcat, aggr2, batch, root, bias, W1, b1, W2, b2)
    return o.reshape(-1)


# speedup vs baseline: 1.8190x; 1.8190x over previous
import functools

import jax
import jax.numpy as jnp
from jax import lax
from jax.experimental import pallas as pl
from jax.experimental.pallas import tpu as pltpu
from jax.experimental.pallas import tpu_sc as plsc

_N = 10000
_E = 160000
_DF = 128
_DE = 16
_DIM = 32
_HID = 32
_NG = 64

_NC = 2
_NS = 16
_NW = _NC * _NS
_LCH = 128
_NCH = _E // _LCH
_NB_NODE = 5
_MB = _N // _NB_NODE
_EB = 4000
_NEB = _E // _EB


def _hcat_body(x_ref, w0_ref, b0_ref, bmat_ref, o_ref):
    h0 = jnp.dot(x_ref[...], w0_ref[...], preferred_element_type=jnp.float32)
    h0 = jnp.maximum(h0 + b0_ref[...], 0.0)
    hB = jnp.dot(h0, bmat_ref[...], preferred_element_type=jnp.float32)
    o_ref[...] = jnp.concatenate(
        [h0, hB, jnp.zeros((_MB, 128 - 2 * _DIM), jnp.float32)], axis=1
    )


def _run_hcat(x, W0, b0, Bmat):
    return pl.pallas_call(
        _hcat_body,
        grid=(_NB_NODE,),
        in_specs=[
            pl.BlockSpec((_MB, _DF), lambda i: (i, 0)),
            pl.BlockSpec((_DF, _DIM), lambda i: (0, 0)),
            pl.BlockSpec((1, _DIM), lambda i: (0, 0)),
            pl.BlockSpec((_DIM, _HID), lambda i: (0, 0)),
        ],
        out_specs=pl.BlockSpec((_MB, 128), lambda i: (i, 0)),
        out_shape=jax.ShapeDtypeStruct((_N, 128), jnp.float32),
    )(x, W0, b0.reshape(1, _DIM), Bmat)


def _gather_body(hcat_hbm, src_hbm, xj_hbm, idx_v, rows_v, sem):
    cid = lax.axis_index("c")
    sid = lax.axis_index("s")
    wid = sid * _NC + cid
    nj = (_NCH // _NW) + (wid < (_NCH % _NW)).astype(jnp.int32)

    def step(j, carry):
        c = wid + j * _NW
        base = pl.multiple_of(c * _LCH, 8)
        pltpu.sync_copy(src_hbm.at[pl.ds(base, _LCH)], idx_v)
        pltpu.async_copy(hcat_hbm.at[idx_v], rows_v, sem).wait()
        pltpu.sync_copy(rows_v, xj_hbm.at[pl.ds(base, _LCH)])
        return carry

    lax.fori_loop(0, nj, step, 0)


def _run_gather(hcat, src):
    mesh = plsc.VectorSubcoreMesh(
        core_axis_name="c", subcore_axis_name="s", num_cores=_NC, num_subcores=_NS
    )
    f = functools.partial(
        pl.kernel,
        out_type=jax.ShapeDtypeStruct((_E, 128), jnp.float32),
        mesh=mesh,
        scratch_types=[
            pltpu.VMEM((_LCH,), jnp.int32),
            pltpu.VMEM((_LCH, 128), jnp.float32),
            pltpu.SemaphoreType.DMA,
        ],
        compiler_params=pltpu.CompilerParams(use_tc_tiling_on_sc=False),
    )(_gather_body)
    return f(hcat, src)


_MSG_DT = jnp.bfloat16


def _msg_body(xj_ref, ea_ref, wcT_ref, o_ref):
    xjT = xj_ref[:, :_DIM].astype(_MSG_DT).T
    eaT = ea_ref[...].astype(_MSG_DT).T
    fT = jnp.concatenate(
        [eaT[k : k + 1, :] * xjT for k in range(_DE)], axis=0
    )
    msgT = jnp.dot(wcT_ref[...], fT, preferred_element_type=jnp.float32)
    msg = msgT.T + xj_ref[:, _DIM : 2 * _DIM]
    q = _EB // 4
    o_ref[...] = jnp.concatenate(
        [msg[g * q : (g + 1) * q] for g in range(4)], axis=1
    )


def _run_msg(xj, ea, WcT):
    return pl.pallas_call(
        _msg_body,
        grid=(_NEB,),
        in_specs=[
            pl.BlockSpec((_EB, 128), lambda i: (i, 0)),
            pl.BlockSpec((_EB, _DE), lambda i: (i, 0)),
            pl.BlockSpec((_HID, _DE * _DIM), lambda i: (0, 0)),
        ],
        out_specs=pl.BlockSpec((_EB // 4, 4 * _HID), lambda i: (i, 0)),
        out_shape=jax.ShapeDtypeStruct((_E // 4, 4 * _HID), jnp.float32),
    )(xj, ea, WcT)


_ZROWS = 125


def _scatter_body(msg_hbm, dst_hbm, out_hbm, idx_v, msg_v, zero_v, shared, sem):
    cid = lax.axis_index("c")
    sid = lax.axis_index("s")
    wid = sid * _NC + cid
    stripe = _N // _NS

    z16 = jnp.zeros((16,), jnp.float32)

    def zrow(i, c):
        zero_v[i, pl.ds(0, 16)] = z16
        zero_v[i, pl.ds(16, 16)] = z16
        return c

    lax.fori_loop(0, _ZROWS, zrow, 0)

    def zcopy(t, c):
        pltpu.sync_copy(zero_v, shared.at[pl.ds(sid * stripe + t * _ZROWS, _ZROWS)])
        return c

    lax.fori_loop(0, stripe // _ZROWS, zcopy, 0)
    plsc.subcore_barrier()

    nj = (_NCH // _NW) + (wid < (_NCH % _NW)).astype(jnp.int32)

    def step(j, carry):
        c = wid + j * _NW
        base = pl.multiple_of(c * _LCH, 8)
        pltpu.sync_copy(dst_hbm.at[pl.ds(base, _LCH)], idx_v)
        pltpu.sync_copy(msg_hbm.at[pl.ds(base, _LCH)], msg_v)
        pltpu.sync_copy(msg_v, shared.at[idx_v], add=True)
        return carry

    lax.fori_loop(0, nj, step, 0)
    plsc.subcore_barrier()

    pltpu.sync_copy(
        shared.at[pl.ds(sid * stripe, stripe)],
        out_hbm.at[cid, pl.ds(sid * stripe, stripe)],
    )


def _run_scatter(msg, dst):
    mesh = plsc.VectorSubcoreMesh(
        core_axis_name="c", subcore_axis_name="s", num_cores=_NC, num_subcores=_NS
    )
    f = functools.partial(
        pl.kernel,
        out_type=jax.ShapeDtypeStruct((_NC, _N, _HID), jnp.float32),
        mesh=mesh,
        scratch_types=[
            pltpu.VMEM((_LCH,), jnp.int32),
            pltpu.VMEM((_LCH, _HID), jnp.float32),
            pltpu.VMEM((_ZROWS, _HID), jnp.float32),
            pltpu.VMEM_SHARED((_N, _HID), jnp.float32),
            pltpu.SemaphoreType.DMA,
        ],
        compiler_params=pltpu.CompilerParams(use_tc_tiling_on_sc=False),
    )(_scatter_body)
    return f(msg, dst)


def _final_body(hcat_ref, a0_ref, a1_ref, b3_ref, root_ref, bias_ref,
                w1_ref, b1_ref, w2_ref, b2_ref, o_ref, u_acc):
    i = pl.program_id(0)
    h0 = hcat_ref[:, :_DIM]
    h = a0_ref[0] + a1_ref[0] + jnp.dot(h0, root_ref[...], preferred_element_type=jnp.float32)
    h = jnp.maximum(h + bias_ref[...], 0.0)
    bids = b3_ref[0]
    oh = (lax.broadcasted_iota(jnp.int32, (_NG, 1), 0) == bids).astype(jnp.float32)
    part = jnp.dot(oh, h, preferred_element_type=jnp.float32)

    @pl.when(i == 0)
    def _():
        u_acc[...] = part

    @pl.when(i > 0)
    def _():
        u_acc[...] += part

    @pl.when(i == pl.num_programs(0) - 1)
    def _():
        u = u_acc[...]
        o1 = jnp.dot(u, w1_ref[...], preferred_element_type=jnp.float32)
        o1 = jnp.maximum(o1 + b1_ref[...], 0.0)
        o_ref[...] = jnp.dot(o1, w2_ref[...], preferred_element_type=jnp.float32) + b2_ref[...]


def _run_final(hcat, aggr2, batch, root, bias, W1, b1, W2, b2):
    batch3 = batch.reshape(_NB_NODE, 1, _MB)
    return pl.pallas_call(
        _final_body,
        grid=(_NB_NODE,),
        in_specs=[
            pl.BlockSpec((_MB, 128), lambda i: (i, 0)),
            pl.BlockSpec((1, _MB, _HID), lambda i: (0, i, 0)),
            pl.BlockSpec((1, _MB, _HID), lambda i: (1, i, 0)),
            pl.BlockSpec((1, 1, _MB), lambda i: (i, 0, 0)),
            pl.BlockSpec((_DIM, _HID), lambda i: (0, 0)),
            pl.BlockSpec((1, _HID), lambda i: (0, 0)),
            pl.BlockSpec((_HID, 16), lambda i: (0, 0)),
            pl.BlockSpec((1, 16), lambda i: (0, 0)),
            pl.BlockSpec((16, 1), lambda i: (0, 0)),
            pl.BlockSpec((1, 1), lambda i: (0, 0)),
        ],
        out_specs=pl.BlockSpec((_NG, 1), lambda i: (0, 0)),
        out_shape=jax.ShapeDtypeStruct((_NG, 1), jnp.float32),
        scratch_shapes=[pltpu.VMEM((_NG, _HID), jnp.float32)],
    )(hcat, aggr2, aggr2, batch3, root, bias.reshape(1, _HID),
      W1, b1.reshape(1, 16), W2, b2.reshape(1, 1))


def kernel(x, edge_index, edge_attr, batch, W0, b0, We, be, root, bias, W1, b1, W2, b2):
    src = edge_index[0]
    dst = edge_index[1]
    Bmat = be.reshape(_DIM, _HID)
    WcT = We.reshape(_DE * _DIM, _HID).T.astype(_MSG_DT)

    dstP = dst.reshape(_NEB, 4, _EB // 4).transpose(0, 2, 1).reshape(-1)

    hcat = _run_hcat(x, W0, b0, Bmat)
    xj = _run_gather(hcat, src)
    msg128 = _run_msg(xj, edge_attr, WcT)
    aggr2 = _run_scatter(msg128.reshape(_E, _HID), dstP)
    o = _run_final(hcat, aggr2, batch, root, bias, W1, b1, W2, b2)
    return o.reshape(-1)
